# Initial kernel scaffold; baseline (speedup 1.0000x reference)
import jax
import jax.numpy as jnp
from jax.experimental import pallas as pl


def kernel(doc, W):
    raise NotImplementedError("write your pallas kernel here")


# speedup vs baseline: 6.3338x; 6.3338x over previous
import functools

import jax
import jax.numpy as jnp
from jax import lax
from jax.experimental import pallas as pl
from jax.experimental.pallas import tpu as pltpu
from jax.experimental.pallas import tpu_sc as plsc

VOCAB = 100000
D = 128
NTOT = 4096 * 200
NC, NS = 2, 16
NW = NC * NS
PER_W = NTOT // NW
CHUNK = 128
NCHUNK = PER_W // CHUNK

_mesh = plsc.VectorSubcoreMesh(
    core_axis_name="c", subcore_axis_name="s", num_cores=NC, num_subcores=NS
)


@functools.partial(
    pl.kernel,
    mesh=_mesh,
    out_type=jax.ShapeDtypeStruct((NTOT, D), jnp.float32),
    scratch_types=[
        pltpu.VMEM((NCHUNK, CHUNK), jnp.int32),
        pltpu.VMEM((CHUNK, D), jnp.float32),
        pltpu.SemaphoreType.DMA,
    ],
)
def _embed_sc(doc_hbm, w_hbm, out_hbm, idx_v, rows_v, gsem):
    wid = lax.axis_index("s") * NC + lax.axis_index("c")
    base = wid * NCHUNK
    pltpu.sync_copy(doc_hbm.at[pl.ds(base, NCHUNK)], idx_v)

    @pl.loop(0, NCHUNK)
    def _chunk(i):
        pltpu.async_copy(w_hbm.at[idx_v.at[i]], rows_v, gsem).wait()
        pltpu.sync_copy(rows_v, out_hbm.at[pl.ds((base + i) * CHUNK, CHUNK)])


def kernel(doc, W):
    idx = doc.reshape(NTOT // CHUNK, CHUNK).astype(jnp.int32)
    out = _embed_sc(idx, W)
    return out.reshape(doc.shape[0], doc.shape[1], D)


# speedup vs baseline: 9.2266x; 1.4567x over previous
import functools

import jax
import jax.numpy as jnp
from jax import lax
from jax.experimental import pallas as pl
from jax.experimental.pallas import tpu as pltpu
from jax.experimental.pallas import tpu_sc as plsc

VOCAB = 100000
D = 128
NTOT = 4096 * 200
NC, NS = 2, 16
NW = NC * NS
PER_W = NTOT // NW
CHUNK = 128
NCHUNK = PER_W // CHUNK
NBUF = 4

_mesh = plsc.VectorSubcoreMesh(
    core_axis_name="c", subcore_axis_name="s", num_cores=NC, num_subcores=NS
)


@functools.partial(
    pl.kernel,
    mesh=_mesh,
    out_type=jax.ShapeDtypeStruct((NTOT, D), jnp.float32),
    scratch_types=[
        pltpu.VMEM((NCHUNK, CHUNK), jnp.int32),
        pltpu.VMEM((NBUF, CHUNK, D), jnp.float32),
        pltpu.SemaphoreType.DMA((NBUF,)),
        pltpu.SemaphoreType.DMA((NBUF,)),
    ],
)
def _embed_sc(doc_hbm, w_hbm, out_hbm, idx_v, rows_v, gsem, ssem):
    wid = lax.axis_index("s") * NC + lax.axis_index("c")
    base = wid * NCHUNK
    pltpu.sync_copy(doc_hbm.at[pl.ds(base, NCHUNK)], idx_v)

    def fire_gather(g, b):
        pltpu.async_copy(w_hbm.at[idx_v.at[g]], rows_v.at[b], gsem.at[b])

    def fire_store(g, b):
        pltpu.async_copy(
            rows_v.at[b], out_hbm.at[pl.ds((base + g) * CHUNK, CHUNK)], ssem.at[b]
        )

    for b in range(NBUF):
        fire_gather(b, b)

    @pl.loop(0, NCHUNK, step=NBUF)
    def _group(i):
        for b in range(NBUF):
            g = i + b
            pltpu.make_async_copy(w_hbm.at[idx_v.at[g]], rows_v.at[b],
                                  gsem.at[b]).wait()
            fire_store(g, b)
            pltpu.make_async_copy(
                rows_v.at[b], out_hbm.at[pl.ds((base + g) * CHUNK, CHUNK)],
                ssem.at[b],
            ).wait()

            @pl.when(g < NCHUNK - NBUF)
            def _():
                fire_gather(g + NBUF, b)


def kernel(doc, W):
    idx = doc.reshape(NTOT // CHUNK, CHUNK).astype(jnp.int32)
    out = _embed_sc(idx, W)
    return out.reshape(doc.shape[0], doc.shape[1], D)


# speedup vs baseline: 9.2809x; 1.0059x over previous
import functools

import jax
import jax.numpy as jnp
from jax import lax
from jax.experimental import pallas as pl
from jax.experimental.pallas import tpu as pltpu
from jax.experimental.pallas import tpu_sc as plsc

VOCAB = 100000
D = 128
NTOT = 4096 * 200
NC, NS = 2, 16
NW = NC * NS
PER_W = NTOT // NW
CHUNK = 128
NCHUNK = PER_W // CHUNK
NBUF = 5

_mesh = plsc.VectorSubcoreMesh(
    core_axis_name="c", subcore_axis_name="s", num_cores=NC, num_subcores=NS
)


@functools.partial(
    pl.kernel,
    mesh=_mesh,
    out_type=jax.ShapeDtypeStruct((NTOT, D), jnp.float32),
    scratch_types=[
        pltpu.VMEM((NCHUNK, CHUNK), jnp.int32),
        pltpu.VMEM((NBUF, CHUNK, D), jnp.float32),
        pltpu.SemaphoreType.DMA((NBUF,)),
        pltpu.SemaphoreType.DMA((NBUF,)),
    ],
)
def _embed_sc(doc_hbm, w_hbm, out_hbm, idx_v, rows_v, gsem, ssem):
    wid = lax.axis_index("s") * NC + lax.axis_index("c")
    base = wid * NCHUNK
    pltpu.sync_copy(doc_hbm.at[pl.ds(base, NCHUNK)], idx_v)

    def fire_gather(g, b):
        pltpu.async_copy(w_hbm.at[idx_v.at[g]], rows_v.at[b], gsem.at[b])

    def fire_store(g, b):
        pltpu.async_copy(
            rows_v.at[b], out_hbm.at[pl.ds((base + g) * CHUNK, CHUNK)], ssem.at[b]
        )

    for b in range(NBUF):
        fire_gather(b, b)

    @pl.loop(0, NCHUNK, step=NBUF)
    def _group(i):
        for b in range(NBUF):
            g = i + b
            pltpu.make_async_copy(w_hbm.at[idx_v.at[g]], rows_v.at[b],
                                  gsem.at[b]).wait()
            fire_store(g, b)
            pltpu.make_async_copy(
                rows_v.at[b], out_hbm.at[pl.ds((base + g) * CHUNK, CHUNK)],
                ssem.at[b],
            ).wait()

            @pl.when(g < NCHUNK - NBUF)
            def _():
                fire_gather(g + NBUF, b)


def kernel(doc, W):
    idx = doc.reshape(NTOT // CHUNK, CHUNK).astype(jnp.int32)
    out = _embed_sc(idx, W)
    return out.reshape(doc.shape[0], doc.shape[1], D)


# speedup vs baseline: 9.2885x; 1.0008x over previous
import functools

import jax
import jax.numpy as jnp
from jax import lax
from jax.experimental import pallas as pl
from jax.experimental.pallas import tpu as pltpu
from jax.experimental.pallas import tpu_sc as plsc

VOCAB = 100000
D = 128
NTOT = 4096 * 200
NC, NS = 2, 16
NW = NC * NS
PER_W = NTOT // NW
CHUNK = 128
NCHUNK = PER_W // CHUNK
NBUF = 5

_mesh = plsc.VectorSubcoreMesh(
    core_axis_name="c", subcore_axis_name="s", num_cores=NC, num_subcores=NS
)


@functools.partial(
    pl.kernel,
    mesh=_mesh,
    out_type=jax.ShapeDtypeStruct((NTOT, D), jnp.float32),
    scratch_types=[
        pltpu.VMEM((NCHUNK, CHUNK), jnp.int32),
        pltpu.VMEM((NBUF, CHUNK, D), jnp.float32),
        pltpu.SemaphoreType.DMA((NBUF,)),
        pltpu.SemaphoreType.DMA((NBUF,)),
    ],
)
def _embed_sc(doc_hbm, w_hbm, out_hbm, idx_v, rows_v, gsem, ssem):
    wid = lax.axis_index("s") * NC + lax.axis_index("c")
    base = wid * NCHUNK
    pltpu.sync_copy(doc_hbm.at[pl.ds(base, NCHUNK)], idx_v)

    def fire_gather(g, b):
        pltpu.async_copy(w_hbm.at[idx_v.at[g]], rows_v.at[b], gsem.at[b])

    def fire_store(g, b):
        pltpu.async_copy(
            rows_v.at[b], out_hbm.at[pl.ds((base + g) * CHUNK, CHUNK)], ssem.at[b]
        )

    def wait_gather(g, b):
        pltpu.make_async_copy(w_hbm.at[idx_v.at[g]], rows_v.at[b],
                              gsem.at[b]).wait()

    def wait_store(g, b):
        pltpu.make_async_copy(
            rows_v.at[b], out_hbm.at[pl.ds((base + g) * CHUNK, CHUNK)],
            ssem.at[b],
        ).wait()

    for b in range(NBUF):
        fire_gather(b, b)

    @pl.loop(0, NCHUNK, step=NBUF)
    def _group(i):
        for b in range(NBUF):
            g = i + b
            wait_gather(g, b)
            fire_store(g, b)
            gp = g - 1
            bp = (b - 1) % NBUF

            @pl.when(jnp.logical_and(gp >= 0, gp < NCHUNK - NBUF))
            def _():
                wait_store(gp, bp)
                fire_gather(gp + NBUF, bp)

    for b in range(NBUF):
        wait_store(NCHUNK - NBUF + b, b)


def kernel(doc, W):
    idx = doc.reshape(NTOT // CHUNK, CHUNK).astype(jnp.int32)
    out = _embed_sc(idx, W)
    return out.reshape(doc.shape[0], doc.shape[1], D)
